# baseline (speedup 1.0000x reference)
import functools

import jax
import jax.numpy as jnp
from jax import lax
from jax.experimental import pallas as pl
from jax.experimental.pallas import tpu as pltpu
from jax.experimental.pallas import tpu_sc as plsc

_NC = 2
_NS = 16
_NW = _NC * _NS

_SC_PARAMS = pltpu.CompilerParams(
    use_tc_tiling_on_sc=True, needs_layout_passes=False)


def _mesh():
    return plsc.VectorSubcoreMesh(
        core_axis_name="c", subcore_axis_name="s",
        num_cores=_NC, num_subcores=_NS)


def _make_fused(m, d, b, rows_per_w, copy_chunk):
    n_copy = rows_per_w // copy_chunk
    tail = m - rows_per_w * _NW
    per_tile = b // _NS
    half = _NS * rows_per_w

    @functools.partial(
        pl.kernel,
        mesh=_mesh(),
        compiler_params=_SC_PARAMS,
        out_type=jax.ShapeDtypeStruct((m * d,), jnp.float32),
        scratch_types=[
            pltpu.VMEM((copy_chunk * d,), jnp.float32),
            pltpu.VMEM((copy_chunk * d,), jnp.float32),
            pltpu.VMEM((128,), jnp.int32),
            pltpu.VMEM((128,), jnp.int32),
            pltpu.VMEM((b // _NS * 64,), jnp.float32),
            pltpu.SemaphoreType.DMA,
            pltpu.SemaphoreType.DMA,
            pltpu.SemaphoreType.DMA,
            pltpu.SemaphoreType.DMA,
            pltpu.SemaphoreType.DMA,
        ],
    )
    def fused(orig_hbm, idx_hbm, wv_hbm, new_hbm, out_ref,
              buf0, buf1, didx, wpv, myrows,
              rs0, rs1, ws0, ws1, ssem):
        core = lax.axis_index("c")
        sub = lax.axis_index("s")
        wid = core * _NS + sub
        base = wid * rows_per_w
        bufs = (buf0, buf1)
        rsems = (rs0, rs1)
        wsems = (ws0, ws1)

        def rd(c):
            return pltpu.make_async_copy(
                orig_hbm.at[pl.ds((base + c * copy_chunk) * d, copy_chunk * d)],
                bufs[c % 2], rsems[c % 2])

        def wr(c):
            return pltpu.make_async_copy(
                bufs[c % 2],
                out_ref.at[pl.ds((base + c * copy_chunk) * d, copy_chunk * d)],
                wsems[c % 2])

        rd(0).start()
        for c in range(n_copy):
            if c + 1 < n_copy:
                if c >= 1:
                    wr(c - 1).wait()
                rd(c + 1).start()
            rd(c).wait()
            wr(c).start()
        if n_copy >= 2:
            wr(n_copy - 2).wait()
        wr(n_copy - 1).wait()

        if tail:
            @pl.when(wid == _NW - 1)
            def _():
                t = pltpu.make_async_copy(
                    orig_hbm.at[pl.ds(rows_per_w * _NW * d, tail * d)],
                    bufs[0].at[pl.ds(0, tail * d)], rsems[0])
                t.start()
                t.wait()
                t2 = pltpu.make_async_copy(
                    bufs[0].at[pl.ds(0, tail * d)],
                    out_ref.at[pl.ds(rows_per_w * _NW * d, tail * d)], wsems[0])
                t2.start()
                t2.wait()

        plsc.subcore_barrier()

        lo = core * half
        hi = half + core * (m - half)

        ebase = sub * per_tile
        lanes = lax.iota(jnp.int32, 16)
        pltpu.sync_copy(new_hbm.at[pl.ds(ebase * d, per_tile * d)], myrows)

        @pl.loop(0, per_tile // 128, init_carry=jnp.int32(0))
        def n_fired(q, carry):
            pltpu.sync_copy(idx_hbm.at[sub].at[q], didx)
            pltpu.sync_copy(wv_hbm.at[sub].at[q], wpv)
            for g in range(8):
                dv = didx[pl.ds(g * 16, 16)]
                wv = wpv[pl.ds(g * 16, 16)]
                mypos = ebase + q * 128 + g * 16 + lanes
                keep = (wv == mypos) & (dv >= lo) & (dv < hi)
                carry = carry + jnp.sum(jnp.where(keep, 1, 0))
                packed = jnp.where(keep, dv, -1)
                for l in range(16):
                    sel = lanes == l
                    dst_s = jnp.sum(jnp.where(sel, packed, 0))
                    src_s = ebase + q * 128 + g * 16 + l

                    loc = (q * 128 + g * 16 + l) * d

                    @pl.when(dst_s >= 0)
                    def _():
                        pltpu.async_copy(
                            myrows.at[pl.ds(loc, d)],
                            out_ref.at[pl.ds(dst_s * d, d)], ssem)
            return carry

        @pl.loop(0, n_fired)
        def _(_i):
            pltpu.make_async_copy(
                myrows.at[pl.ds(0, d)],
                out_ref.at[pl.ds(0, d)], ssem).wait()

    return fused


def kernel(mat_orig, indices, mat_new):
    m, d = mat_orig.shape
    b = indices.shape[0]
    rows_per_w = (m // _NW) // 8 * 8
    copy_chunk = 248
    assert rows_per_w % copy_chunk == 0

    idx = indices.astype(jnp.int32)
    pos = jnp.arange(b, dtype=jnp.int32)
    wpos = jnp.full((m,), -1, jnp.int32).at[idx].max(pos)
    wvals = wpos[idx]

    per_tile = b // _NS
    idx3 = idx.reshape(_NS, per_tile // 128, 128)
    wv3 = wvals.reshape(_NS, per_tile // 128, 128)

    fused = _make_fused(m, d, b, rows_per_w, copy_chunk)
    out = fused(mat_orig.reshape(m * d), idx3, wv3, mat_new.reshape(b * d))
    return out.reshape(m, d)


# speedup vs baseline: 1.1018x; 1.1018x over previous
import functools

import jax
import jax.numpy as jnp
from jax import lax
from jax.experimental import pallas as pl
from jax.experimental.pallas import tpu as pltpu
from jax.experimental.pallas import tpu_sc as plsc

_NC = 2
_NS = 16
_NW = _NC * _NS

_SC_PARAMS = pltpu.CompilerParams(
    use_tc_tiling_on_sc=True, needs_layout_passes=False)


def _mesh():
    return plsc.VectorSubcoreMesh(
        core_axis_name="c", subcore_axis_name="s",
        num_cores=_NC, num_subcores=_NS)


def _make_fused(m, d, b, rows_per_w, copy_chunk):
    n_copy = rows_per_w // copy_chunk
    tail = m - rows_per_w * _NW
    per_tile = b // _NS
    half = _NS * rows_per_w

    @functools.partial(
        pl.kernel,
        mesh=_mesh(),
        compiler_params=_SC_PARAMS,
        out_type=jax.ShapeDtypeStruct((m, d), jnp.float32),
        scratch_types=[
            pltpu.VMEM((copy_chunk, d), jnp.float32),
            pltpu.VMEM((copy_chunk, d), jnp.float32),
            pltpu.VMEM((128,), jnp.int32),
            pltpu.VMEM((128,), jnp.int32),
            pltpu.SMEM((b // _NS,), jnp.int32),
            pltpu.SemaphoreType.DMA,
            pltpu.SemaphoreType.DMA,
            pltpu.SemaphoreType.DMA,
            pltpu.SemaphoreType.DMA,
            pltpu.SemaphoreType.DMA,
        ],
    )
    def fused(orig_hbm, idx_hbm, wv_hbm, new_hbm, out_ref,
              buf0, buf1, didx, wpv, spad,
              rs0, rs1, ws0, ws1, ssem):
        core = lax.axis_index("c")
        sub = lax.axis_index("s")
        wid = core * _NS + sub
        base = wid * rows_per_w
        bufs = (buf0, buf1)
        rsems = (rs0, rs1)
        wsems = (ws0, ws1)

        def rd(c):
            return pltpu.make_async_copy(
                orig_hbm.at[pl.ds(base + c * copy_chunk, copy_chunk)],
                bufs[c % 2], rsems[c % 2])

        def wr(c):
            return pltpu.make_async_copy(
                bufs[c % 2],
                out_ref.at[pl.ds(base + c * copy_chunk, copy_chunk)],
                wsems[c % 2])

        ebase = sub * per_tile
        lanes = lax.iota(jnp.int32, 16)
        lo = core * half
        hi = half + core * (m - half)

        def sweep(q, cnt):
            pltpu.sync_copy(idx_hbm.at[sub].at[q], didx)
            pltpu.sync_copy(wv_hbm.at[sub].at[q], wpv)
            for g in range(8):
                dv = didx[pl.ds(g * 16, 16)]
                wv = wpv[pl.ds(g * 16, 16)]
                mypos = ebase + q * 128 + g * 16 + lanes
                keep = (wv == mypos) & (dv >= lo) & (dv < hi)
                packed = jnp.where(keep, dv * 1024 + (q * 128 + g * 16 + lanes),
                                   -1)

                @pl.loop(0, 16, init_carry=cnt)
                def cnt_new(l, cc):
                    v = jnp.sum(jnp.where(lanes == l, packed, 0))

                    def put():
                        spad[cc] = v
                        return cc + 1

                    return lax.cond(v >= 0, put, lambda: cc)

                cnt = cnt_new
            return cnt

        n_cnk = per_tile // 128
        cnt = jnp.int32(0)
        rd(0).start()
        for c in range(n_copy):
            if c + 1 < n_copy:
                if c >= 1:
                    wr(c - 1).wait()
                rd(c + 1).start()
            if c < n_cnk:
                cnt = sweep(c, cnt)
            rd(c).wait()
            wr(c).start()
        if n_copy >= 2:
            wr(n_copy - 2).wait()
        wr(n_copy - 1).wait()

        if tail:
            @pl.when(wid == _NW - 1)
            def _():
                t = pltpu.make_async_copy(
                    orig_hbm.at[pl.ds(rows_per_w * _NW, tail)],
                    bufs[0].at[pl.ds(0, tail)], rsems[0])
                t.start()
                t.wait()
                t2 = pltpu.make_async_copy(
                    bufs[0].at[pl.ds(0, tail)],
                    out_ref.at[pl.ds(rows_per_w * _NW, tail)], wsems[0])
                t2.start()
                t2.wait()

        plsc.subcore_barrier()

        @pl.loop(0, cnt)
        def _(i):
            v = spad[i]
            dst_s = v // 1024
            src_s = ebase + v % 1024
            pltpu.async_copy(
                new_hbm.at[pl.ds(src_s, 1)],
                out_ref.at[pl.ds(dst_s, 1)], ssem)

        @pl.loop(0, cnt)
        def _(_i):
            pltpu.make_async_copy(
                new_hbm.at[pl.ds(0, 1)], out_ref.at[pl.ds(0, 1)], ssem).wait()

    return fused


def kernel(mat_orig, indices, mat_new):
    m, d = mat_orig.shape
    b = indices.shape[0]
    rows_per_w = (m // _NW) // 8 * 8
    copy_chunk = 248
    assert rows_per_w % copy_chunk == 0

    idx = indices.astype(jnp.int32)
    pos = jnp.arange(b, dtype=jnp.int32)
    wpos = jnp.full((m,), -1, jnp.int32).at[idx].max(pos)
    wvals = wpos[idx]

    per_tile = b // _NS
    idx3 = idx.reshape(_NS, per_tile // 128, 128)
    wv3 = wvals.reshape(_NS, per_tile // 128, 128)

    fused = _make_fused(m, d, b, rows_per_w, copy_chunk)
    return fused(mat_orig, idx3, wv3, mat_new)


# speedup vs baseline: 1.1058x; 1.0037x over previous
import functools

import jax
import jax.numpy as jnp
from jax import lax
from jax.experimental import pallas as pl
from jax.experimental.pallas import tpu as pltpu
from jax.experimental.pallas import tpu_sc as plsc

_NC = 2
_NS = 16
_NW = _NC * _NS

_SC_PARAMS = pltpu.CompilerParams(
    use_tc_tiling_on_sc=True, needs_layout_passes=False)


def _mesh():
    return plsc.VectorSubcoreMesh(
        core_axis_name="c", subcore_axis_name="s",
        num_cores=_NC, num_subcores=_NS)


def _make_fused(m, d, b, rows_per_w, copy_chunk):
    n_copy = rows_per_w // copy_chunk
    tail = m - rows_per_w * _NW
    per_tile = b // _NS
    half = _NS * rows_per_w

    @functools.partial(
        pl.kernel,
        mesh=_mesh(),
        compiler_params=_SC_PARAMS,
        out_type=jax.ShapeDtypeStruct((m, d), jnp.float32),
        scratch_types=[
            pltpu.VMEM((copy_chunk, d), jnp.float32),
            pltpu.VMEM((copy_chunk, d), jnp.float32),
            pltpu.VMEM((128,), jnp.int32),
            pltpu.VMEM((128,), jnp.int32),
            pltpu.SemaphoreType.DMA,
            pltpu.SemaphoreType.DMA,
            pltpu.SemaphoreType.DMA,
            pltpu.SemaphoreType.DMA,
            pltpu.SemaphoreType.DMA,
        ],
    )
    def fused(orig_hbm, idx_hbm, wv_hbm, new_hbm, out_ref,
              buf0, buf1, didx, wpv,
              rs0, rs1, ws0, ws1, ssem):
        core = lax.axis_index("c")
        sub = lax.axis_index("s")
        wid = core * _NS + sub
        base = wid * rows_per_w
        bufs = (buf0, buf1)
        rsems = (rs0, rs1)
        wsems = (ws0, ws1)

        def rd(c):
            return pltpu.make_async_copy(
                orig_hbm.at[pl.ds(base + c * copy_chunk, copy_chunk)],
                bufs[c % 2], rsems[c % 2])

        def wr(c):
            return pltpu.make_async_copy(
                bufs[c % 2],
                out_ref.at[pl.ds(base + c * copy_chunk, copy_chunk)],
                wsems[c % 2])

        rd(0).start()
        for c in range(n_copy):
            if c + 1 < n_copy:
                if c >= 1:
                    wr(c - 1).wait()
                rd(c + 1).start()
            rd(c).wait()
            wr(c).start()
        if n_copy >= 2:
            wr(n_copy - 2).wait()
        wr(n_copy - 1).wait()

        if tail:
            @pl.when(wid == _NW - 1)
            def _():
                t = pltpu.make_async_copy(
                    orig_hbm.at[pl.ds(rows_per_w * _NW, tail)],
                    bufs[0].at[pl.ds(0, tail)], rsems[0])
                t.start()
                t.wait()
                t2 = pltpu.make_async_copy(
                    bufs[0].at[pl.ds(0, tail)],
                    out_ref.at[pl.ds(rows_per_w * _NW, tail)], wsems[0])
                t2.start()
                t2.wait()

        plsc.subcore_barrier()

        lo = core * half
        hi = half + core * (m - half)

        ebase = sub * per_tile
        lanes = lax.iota(jnp.int32, 16)

        @pl.loop(0, per_tile // 128, init_carry=jnp.int32(0))
        def n_fired(q, carry):
            pltpu.sync_copy(idx_hbm.at[sub].at[q], didx)
            pltpu.sync_copy(wv_hbm.at[sub].at[q], wpv)
            for g in range(8):
                dv = didx[pl.ds(g * 16, 16)]
                wv = wpv[pl.ds(g * 16, 16)]
                mypos = ebase + q * 128 + g * 16 + lanes
                keep = (wv == mypos) & (dv >= lo) & (dv < hi)
                carry = carry + jnp.sum(jnp.where(keep, 1, 0))
                packed = jnp.where(keep, dv, -1)
                for l in range(16):
                    sel = lanes == l
                    dst_s = jnp.sum(jnp.where(sel, packed, 0))
                    src_s = ebase + q * 128 + g * 16 + l

                    @pl.when(dst_s >= 0)
                    def _():
                        pltpu.async_copy(
                            new_hbm.at[pl.ds(src_s, 1)],
                            out_ref.at[pl.ds(dst_s, 1)], ssem)
            return carry

        @pl.loop(0, n_fired)
        def _(_i):
            pltpu.make_async_copy(
                new_hbm.at[pl.ds(0, 1)], out_ref.at[pl.ds(0, 1)], ssem).wait()

    return fused


def kernel(mat_orig, indices, mat_new):
    m, d = mat_orig.shape
    b = indices.shape[0]
    rows_per_w = (m // _NW) // 8 * 8
    copy_chunk = 248
    assert rows_per_w % copy_chunk == 0

    idx = indices.astype(jnp.int32)
    pos = jnp.arange(b, dtype=jnp.int32)
    wpos = jnp.full((m,), -1, jnp.int32).at[idx].max(pos)
    wvals = wpos[idx]

    per_tile = b // _NS
    idx3 = idx.reshape(_NS, per_tile // 128, 128)
    wv3 = wvals.reshape(_NS, per_tile // 128, 128)

    fused = _make_fused(m, d, b, rows_per_w, copy_chunk)
    return fused(mat_orig, idx3, wv3, mat_new)
